# speedup vs baseline: 4.3387x; 4.3387x over previous
import functools

import jax
import jax.numpy as jnp
from jax import lax
from jax.experimental import pallas as pl
from jax.experimental.pallas import tpu as pltpu
from jax.experimental.pallas import tpu_sc as plsc

N_NODES = 10000
D = 128
N_GRAPHS = 64

NC = 2
NS = 16
NW = NC * NS
CHUNK = 128
ROWS_PER_TILE = 640
N_PAD = NS * ROWS_PER_TILE
BLK = 1024


def _sc_aggregate(px, z640, src3, dst3, n_chunks):
  mesh = plsc.VectorSubcoreMesh(core_axis_name="c", subcore_axis_name="s")

  @functools.partial(
      pl.kernel,
      out_type=jax.ShapeDtypeStruct((NC, N_PAD, D), jnp.float32),
      mesh=mesh,
      scratch_types=[
          pltpu.MemorySpace.VMEM_SHARED((N_PAD, D), jnp.float32),
          pltpu.MemorySpace.VMEM((n_chunks, CHUNK), jnp.int32),
          pltpu.MemorySpace.VMEM((n_chunks, CHUNK), jnp.int32),
          pltpu.MemorySpace.VMEM((CHUNK, D), jnp.float32),
          pltpu.SemaphoreType.DMA,
      ],
  )
  def agg(px_hbm, z_hbm, src_hbm, dst_hbm, out_hbm,
          acc_sh, src_v, dst_v, rows_v, sem):
    c = lax.axis_index("c")
    s = lax.axis_index("s")
    wid = s * NC + c
    r0 = s * ROWS_PER_TILE

    @pl.when(c == 0)
    def _():
      pltpu.sync_copy(px_hbm.at[pl.ds(r0, ROWS_PER_TILE)],
                      acc_sh.at[pl.ds(r0, ROWS_PER_TILE)])

    @pl.when(c == 1)
    def _():
      pltpu.sync_copy(z_hbm, acc_sh.at[pl.ds(r0, ROWS_PER_TILE)])

    pltpu.sync_copy(src_hbm.at[wid], src_v)
    pltpu.sync_copy(dst_hbm.at[wid], dst_v)
    plsc.subcore_barrier()

    def step(j, carry):
      pltpu.async_copy(px_hbm.at[src_v.at[j]], rows_v, sem).wait()
      pltpu.sync_copy(rows_v, acc_sh.at[dst_v.at[j]], add=True)
      return carry

    lax.fori_loop(0, n_chunks, step, 0)
    plsc.subcore_barrier()

    pltpu.sync_copy(acc_sh.at[pl.ds(r0, ROWS_PER_TILE)],
                    out_hbm.at[c].at[pl.ds(r0, ROWS_PER_TILE)])

  return agg(px, z640, src3, dst3)


def _mlp_body(a0_ref, a1_ref, wa_ref, ba_ref, wb_ref, bb_ref, out_ref):
  h = a0_ref[...] + a1_ref[...]
  h = jnp.maximum(
      jnp.dot(h, wa_ref[...], preferred_element_type=jnp.float32)
      + ba_ref[...], 0.0)
  h = jnp.maximum(
      jnp.dot(h, wb_ref[...], preferred_element_type=jnp.float32)
      + bb_ref[...], 0.0)
  out_ref[...] = h


def _mlp(a0, a1, Wa, ba, Wb, bb):
  n_blocks = N_PAD // BLK
  return pl.pallas_call(
      _mlp_body,
      grid=(n_blocks,),
      in_specs=[
          pl.BlockSpec((BLK, D), lambda i: (i, 0)),
          pl.BlockSpec((BLK, D), lambda i: (i, 0)),
          pl.BlockSpec((D, D), lambda i: (0, 0)),
          pl.BlockSpec((1, D), lambda i: (0, 0)),
          pl.BlockSpec((D, D), lambda i: (0, 0)),
          pl.BlockSpec((1, D), lambda i: (0, 0)),
      ],
      out_specs=pl.BlockSpec((BLK, D), lambda i: (i, 0)),
      out_shape=jax.ShapeDtypeStruct((N_PAD, D), jnp.float32),
  )(a0, a1, Wa, ba.reshape(1, D), Wb, bb.reshape(1, D))


def _mlp_pool_body(a0_ref, a1_ref, wa_ref, ba_ref, wb_ref, bb_ref, b_ref,
                   pool_ref):
  h = a0_ref[...] + a1_ref[...]
  h = jnp.maximum(
      jnp.dot(h, wa_ref[...], preferred_element_type=jnp.float32)
      + ba_ref[...], 0.0)
  h = jnp.maximum(
      jnp.dot(h, wb_ref[...], preferred_element_type=jnp.float32)
      + bb_ref[...], 0.0)
  seg = b_ref[0, 0, :]
  onehot = (lax.broadcasted_iota(jnp.int32, (N_GRAPHS, BLK), 0)
            == seg[None, :]).astype(jnp.float32)

  @pl.when(pl.program_id(0) == 0)
  def _():
    pool_ref[...] = jnp.zeros_like(pool_ref)

  pool_ref[...] += jnp.dot(onehot, h, preferred_element_type=jnp.float32)


def _mlp_pool(a0, a1, Wa, ba, Wb, bb, batch3):
  n_blocks = N_PAD // BLK
  return pl.pallas_call(
      _mlp_pool_body,
      grid=(n_blocks,),
      in_specs=[
          pl.BlockSpec((BLK, D), lambda i: (i, 0)),
          pl.BlockSpec((BLK, D), lambda i: (i, 0)),
          pl.BlockSpec((D, D), lambda i: (0, 0)),
          pl.BlockSpec((1, D), lambda i: (0, 0)),
          pl.BlockSpec((D, D), lambda i: (0, 0)),
          pl.BlockSpec((1, D), lambda i: (0, 0)),
          pl.BlockSpec((1, 1, BLK), lambda i: (i, 0, 0)),
      ],
      out_specs=pl.BlockSpec((N_GRAPHS, D), lambda i: (0, 0)),
      out_shape=jax.ShapeDtypeStruct((N_GRAPHS, D), jnp.float32),
  )(a0, a1, Wa, ba.reshape(1, D), Wb, bb.reshape(1, D), batch3)


def _head_body(p_ref, w1_ref, b1_ref, w2_ref, b2_ref, out_ref):
  h = jnp.maximum(
      jnp.dot(p_ref[...], w1_ref[...], preferred_element_type=jnp.float32)
      + b1_ref[...], 0.0)
  out_ref[...] = (
      jnp.dot(h, w2_ref[...], preferred_element_type=jnp.float32)
      + b2_ref[...])


def _head(pooled, Wl1, bl1, Wl2p, bl2b):
  return pl.pallas_call(
      _head_body,
      in_specs=[
          pl.BlockSpec((N_GRAPHS, D), lambda: (0, 0)),
          pl.BlockSpec((D, D), lambda: (0, 0)),
          pl.BlockSpec((1, D), lambda: (0, 0)),
          pl.BlockSpec((D, D), lambda: (0, 0)),
          pl.BlockSpec((1, D), lambda: (0, 0)),
      ],
      out_specs=pl.BlockSpec((N_GRAPHS, D), lambda: (0, 0)),
      out_shape=jax.ShapeDtypeStruct((N_GRAPHS, D), jnp.float32),
  )(pooled, Wl1, bl1.reshape(1, D), Wl2p, bl2b)


def kernel(x, edge_index, batch, W1a, b1a, W1b, b1b, W2a, b2a, W2b, b2b,
           Wl1, bl1, Wl2, bl2):
  n_edges = edge_index.shape[1]
  per_dma = NW * CHUNK
  n_chunks = -(-n_edges // per_dma)
  ep = n_chunks * per_dma

  src = edge_index[0].astype(jnp.int32)
  dst = edge_index[1].astype(jnp.int32)
  pad = ep - n_edges
  src_p = jnp.concatenate([src, jnp.zeros((pad,), jnp.int32)])
  dst_p = jnp.concatenate([dst, jnp.full((pad,), N_NODES, jnp.int32)])
  src3 = src_p.reshape(NW, n_chunks, CHUNK)
  dst3 = dst_p.reshape(NW, n_chunks, CHUNK)

  px = jnp.concatenate(
      [x, jnp.zeros((N_PAD - N_NODES, D), jnp.float32)], axis=0)
  z640 = jnp.zeros((ROWS_PER_TILE, D), jnp.float32)

  batch_p = jnp.concatenate([
      batch.astype(jnp.int32),
      jnp.full((N_PAD - N_NODES,), N_GRAPHS, jnp.int32)
  ])
  batch3 = batch_p.reshape(N_PAD // BLK, 1, BLK)

  acc1 = _sc_aggregate(px, z640, src3, dst3, n_chunks)
  h1 = _mlp(acc1[0], acc1[1], W1a, b1a, W1b, b1b)
  acc2 = _sc_aggregate(h1, z640, src3, dst3, n_chunks)
  pooled = _mlp_pool(acc2[0], acc2[1], W2a, b2a, W2b, b2b, batch3)

  Wl2p = jnp.pad(Wl2, ((0, 0), (0, D - Wl2.shape[1])))
  bl2b = jnp.broadcast_to(bl2.reshape(1, 1), (1, D))
  out = _head(pooled, Wl1, bl1, Wl2p, bl2b)
  return out[:, :1]
